# baseline (speedup 1.0000x reference)
import jax
import jax.numpy as jnp
from jax import lax
from jax.experimental import pallas as pl
from jax.experimental.pallas import tpu as pltpu

NQ = 6
K = 1024
C = 256
BB = 64
TT = 576
NROWS = BB * TT
R = 256
NBLK = NROWS // R


def _vq_kernel(xf_ref, cb_ref, cbsq_ref, qo_ref, idx_ref, loss_ref, perp_ref,
               counts):
    i = pl.program_id(0)
    residual = xf_ref[...]
    qacc = jnp.zeros((R, C), jnp.float32)
    ii = lax.broadcasted_iota(jnp.int32, (R, K), 1)
    idx_cols = []
    loss_rows = []
    count_rows = []
    for q in range(NQ):
        cb = cb_ref[q]
        rsq = jnp.sum(residual * residual, axis=1, keepdims=True)
        cross = lax.dot_general(residual, cb, (((1,), (1,)), ((), ())),
                                preferred_element_type=jnp.float32)
        d = rsq - 2.0 * cross + cbsq_ref[q]
        m = jnp.min(d, axis=1, keepdims=True)
        idxc = jnp.min(jnp.where(d == m, ii, K), axis=1, keepdims=True)
        oh = (ii == idxc).astype(jnp.float32)
        x_d = lax.dot_general(oh, cb, (((1,), (0,)), ((), ())),
                              preferred_element_type=jnp.float32,
                              precision=lax.Precision.HIGHEST)
        counts_q = jnp.sum(oh, axis=0, keepdims=True)
        residual = residual - x_d
        sq = jnp.sum(residual * residual)
        qacc = qacc + x_d
        idx_cols.append(idxc)
        loss_rows.append(jnp.full((1, 128), sq, jnp.float32))
        count_rows.append(counts_q)

    qo_ref[...] = qacc
    idx_ref[...] = jnp.concatenate(
        idx_cols + [jnp.zeros((R, 8 - NQ), jnp.int32)], axis=1)
    loss_blk = jnp.concatenate(
        loss_rows + [jnp.zeros((8 - NQ, 128), jnp.float32)], axis=0)
    counts_blk = jnp.concatenate(
        count_rows + [jnp.zeros((8 - NQ, K), jnp.float32)], axis=0)

    @pl.when(i == 0)
    def _():
        counts[...] = counts_blk
        loss_ref[...] = loss_blk

    @pl.when(i > 0)
    def _():
        counts[...] += counts_blk
        loss_ref[...] += loss_blk

    @pl.when(i == NBLK - 1)
    def _():
        prob = counts[...] * (1.0 / NROWS)
        plog = prob * jnp.log(prob + 1e-7)
        s = jnp.sum(plog, axis=1, keepdims=True)
        perp_ref[...] = jnp.broadcast_to(jnp.exp(-s), (8, 128))


def kernel(x, codebooks):
    xf = x.transpose(0, 2, 1).reshape(NROWS, C)
    cbsq = jnp.sum(codebooks ** 2, axis=-1).reshape(NQ, 1, K)
    qo_flat, idx8, loss8, perp8 = pl.pallas_call(
        _vq_kernel,
        grid=(NBLK,),
        in_specs=[
            pl.BlockSpec((R, C), lambda i: (i, 0)),
            pl.BlockSpec((NQ, K, C), lambda i: (0, 0, 0)),
            pl.BlockSpec((NQ, 1, K), lambda i: (0, 0, 0)),
        ],
        out_specs=[
            pl.BlockSpec((R, C), lambda i: (i, 0)),
            pl.BlockSpec((R, 8), lambda i: (i, 0)),
            pl.BlockSpec((8, 128), lambda i: (0, 0)),
            pl.BlockSpec((8, 128), lambda i: (0, 0)),
        ],
        out_shape=[
            jax.ShapeDtypeStruct((NROWS, C), jnp.float32),
            jax.ShapeDtypeStruct((NROWS, 8), jnp.int32),
            jax.ShapeDtypeStruct((8, 128), jnp.float32),
            jax.ShapeDtypeStruct((8, 128), jnp.float32),
        ],
        scratch_shapes=[pltpu.VMEM((8, K), jnp.float32)],
    )(xf, codebooks, cbsq)
    qo = qo_flat.reshape(BB, TT, C).transpose(0, 2, 1)
    indices = idx8[:, :NQ].reshape(BB, TT, NQ)
    losses = loss8[:NQ, 0] / (NROWS * C)
    perp = perp8[:NQ, 0]
    return qo, indices, losses, perp


# speedup vs baseline: 1.5009x; 1.5009x over previous
import jax
import jax.numpy as jnp
from jax import lax
from jax.experimental import pallas as pl
from jax.experimental.pallas import tpu as pltpu

NQ = 6
K = 1024
C = 256
BB = 64
TT = 576
NROWS = BB * TT
R = 256
NBLK = NROWS // R


def _vq_kernel(xf_ref, cb_ref, cbsq_ref, cb3_ref,
               qo_ref, idx_ref, loss_ref, perp_ref, counts):
    i = pl.program_id(0)
    residual = xf_ref[...]
    qacc = jnp.zeros((R, C), jnp.float32)
    ii = lax.broadcasted_iota(jnp.int32, (R, K), 1)
    ii3 = lax.broadcasted_iota(jnp.int32, (R, 3 * K), 1)
    idx_cols = []
    loss_rows = []
    count_rows = []
    for q in range(NQ):
        cb = cb_ref[q]
        rsq = jnp.sum(residual * residual, axis=1, keepdims=True)
        cross = lax.dot_general(residual, cb, (((1,), (1,)), ((), ())),
                                preferred_element_type=jnp.float32)
        d = rsq - 2.0 * cross + cbsq_ref[q]
        m = jnp.min(d, axis=1, keepdims=True)
        idxc = jnp.min(jnp.where(d == m, ii, K), axis=1, keepdims=True)
        oh3f = ((ii3 & (K - 1)) == idxc).astype(jnp.float32)
        oh3 = oh3f.astype(jnp.bfloat16)
        dn = (((1,), (0,)), ((), ()))
        x_d = lax.dot_general(oh3, cb3_ref[q], dn,
                              preferred_element_type=jnp.float32)
        counts_q = jnp.sum(oh3f[:, :K], axis=0, keepdims=True)
        residual = residual - x_d
        sq = jnp.sum(residual * residual)
        qacc = qacc + x_d
        idx_cols.append(idxc)
        loss_rows.append(jnp.full((1, 128), sq, jnp.float32))
        count_rows.append(counts_q)

    qo_ref[...] = qacc
    idx_ref[...] = jnp.concatenate(
        idx_cols + [jnp.zeros((R, 8 - NQ), jnp.int32)], axis=1)
    loss_blk = jnp.concatenate(
        loss_rows + [jnp.zeros((8 - NQ, 128), jnp.float32)], axis=0)
    counts_blk = jnp.concatenate(
        count_rows + [jnp.zeros((8 - NQ, K), jnp.float32)], axis=0)

    @pl.when(i == 0)
    def _():
        counts[...] = counts_blk
        loss_ref[...] = loss_blk

    @pl.when(i > 0)
    def _():
        counts[...] += counts_blk
        loss_ref[...] += loss_blk

    @pl.when(i == NBLK - 1)
    def _():
        prob = counts[...] * (1.0 / NROWS)
        plog = prob * jnp.log(prob + 1e-7)
        s = jnp.sum(plog, axis=1, keepdims=True)
        perp_ref[...] = jnp.broadcast_to(jnp.exp(-s), (8, 128))


def kernel(x, codebooks):
    xf = x.transpose(0, 2, 1).reshape(NROWS, C)
    cbsq = jnp.sum(codebooks ** 2, axis=-1).reshape(NQ, 1, K)
    bits = lax.bitcast_convert_type(codebooks, jnp.int32)
    hi_f = lax.bitcast_convert_type(bits & jnp.int32(-65536), jnp.float32)
    rem = codebooks - hi_f
    rbits = lax.bitcast_convert_type(rem, jnp.int32)
    mid_f = lax.bitcast_convert_type(rbits & jnp.int32(-65536), jnp.float32)
    lo_f = rem - mid_f
    cb_hi = hi_f.astype(jnp.bfloat16)
    cb_mid = mid_f.astype(jnp.bfloat16)
    cb_lo = lo_f.astype(jnp.bfloat16)
    cb3 = jnp.concatenate([cb_lo, cb_mid, cb_hi], axis=1)
    qo_flat, idx8, loss8, perp8 = pl.pallas_call(
        _vq_kernel,
        grid=(NBLK,),
        in_specs=[
            pl.BlockSpec((R, C), lambda i: (i, 0)),
            pl.BlockSpec((NQ, K, C), lambda i: (0, 0, 0)),
            pl.BlockSpec((NQ, 1, K), lambda i: (0, 0, 0)),
            pl.BlockSpec((NQ, 3 * K, C), lambda i: (0, 0, 0)),
        ],
        out_specs=[
            pl.BlockSpec((R, C), lambda i: (i, 0)),
            pl.BlockSpec((R, 8), lambda i: (i, 0)),
            pl.BlockSpec((8, 128), lambda i: (0, 0)),
            pl.BlockSpec((8, 128), lambda i: (0, 0)),
        ],
        out_shape=[
            jax.ShapeDtypeStruct((NROWS, C), jnp.float32),
            jax.ShapeDtypeStruct((NROWS, 8), jnp.int32),
            jax.ShapeDtypeStruct((8, 128), jnp.float32),
            jax.ShapeDtypeStruct((8, 128), jnp.float32),
        ],
        scratch_shapes=[pltpu.VMEM((8, K), jnp.float32)],
    )(xf, codebooks, cbsq, cb3)
    qo = qo_flat.reshape(BB, TT, C).transpose(0, 2, 1)
    indices = idx8[:, :NQ].reshape(BB, TT, NQ)
    losses = loss8[:NQ, 0] / (NROWS * C)
    perp = perp8[:NQ, 0]
    return qo, indices, losses, perp


# speedup vs baseline: 1.5584x; 1.0383x over previous
import jax
import jax.numpy as jnp
from jax import lax
from jax.experimental import pallas as pl
from jax.experimental.pallas import tpu as pltpu

NQ = 6
K = 1024
C = 256
BB = 64
TT = 576
NROWS = BB * TT
R = 256
NBLK = NROWS // R


def _vq_kernel(xf_ref, cb_ref, cbsq_ref, cb3_ref,
               qo_ref, idx_ref, loss_ref, perp_ref, counts):
    i = pl.program_id(0)
    residual = xf_ref[...]
    qacc = jnp.zeros((R, C), jnp.float32)
    ii = lax.broadcasted_iota(jnp.int32, (R, K), 1)
    idx_cols = []
    loss_rows = []
    count_rows = []
    for q in range(NQ):
        cb = cb_ref[q]
        rsq = jnp.sum(residual * residual, axis=1, keepdims=True)
        cross = lax.dot_general(residual, cb, (((1,), (1,)), ((), ())),
                                preferred_element_type=jnp.float32)
        d = rsq - 2.0 * cross + cbsq_ref[q]
        idxc = jnp.argmin(d, axis=1, keepdims=True).astype(jnp.int32)
        ohf = (ii == idxc).astype(jnp.float32)
        oh = ohf.astype(jnp.bfloat16)
        oh3 = jnp.concatenate([oh, oh, oh], axis=1)
        dn = (((1,), (0,)), ((), ()))
        x_d = lax.dot_general(oh3, cb3_ref[q], dn,
                              preferred_element_type=jnp.float32)
        counts_q = jnp.sum(ohf, axis=0, keepdims=True)
        residual = residual - x_d
        sq = jnp.sum(residual * residual)
        qacc = qacc + x_d
        idx_cols.append(idxc)
        loss_rows.append(jnp.full((1, 128), sq, jnp.float32))
        count_rows.append(counts_q)

    qo_ref[...] = qacc
    idx_ref[...] = jnp.concatenate(
        idx_cols + [jnp.zeros((R, 8 - NQ), jnp.int32)], axis=1)
    loss_blk = jnp.concatenate(
        loss_rows + [jnp.zeros((8 - NQ, 128), jnp.float32)], axis=0)
    counts_blk = jnp.concatenate(
        count_rows + [jnp.zeros((8 - NQ, K), jnp.float32)], axis=0)

    @pl.when(i == 0)
    def _():
        counts[...] = counts_blk
        loss_ref[...] = loss_blk

    @pl.when(i > 0)
    def _():
        counts[...] += counts_blk
        loss_ref[...] += loss_blk

    @pl.when(i == NBLK - 1)
    def _():
        prob = counts[...] * (1.0 / NROWS)
        plog = prob * jnp.log(prob + 1e-7)
        s = jnp.sum(plog, axis=1, keepdims=True)
        perp_ref[...] = jnp.broadcast_to(jnp.exp(-s), (8, 128))


def kernel(x, codebooks):
    xf = x.transpose(0, 2, 1).reshape(NROWS, C)
    cbsq = jnp.sum(codebooks ** 2, axis=-1).reshape(NQ, 1, K)
    bits = lax.bitcast_convert_type(codebooks, jnp.int32)
    hi_f = lax.bitcast_convert_type(bits & jnp.int32(-65536), jnp.float32)
    rem = codebooks - hi_f
    rbits = lax.bitcast_convert_type(rem, jnp.int32)
    mid_f = lax.bitcast_convert_type(rbits & jnp.int32(-65536), jnp.float32)
    lo_f = rem - mid_f
    cb_hi = hi_f.astype(jnp.bfloat16)
    cb_mid = mid_f.astype(jnp.bfloat16)
    cb_lo = lo_f.astype(jnp.bfloat16)
    cb3 = jnp.concatenate([cb_lo, cb_mid, cb_hi], axis=1)
    qo_flat, idx8, loss8, perp8 = pl.pallas_call(
        _vq_kernel,
        grid=(NBLK,),
        in_specs=[
            pl.BlockSpec((R, C), lambda i: (i, 0)),
            pl.BlockSpec((NQ, K, C), lambda i: (0, 0, 0)),
            pl.BlockSpec((NQ, 1, K), lambda i: (0, 0, 0)),
            pl.BlockSpec((NQ, 3 * K, C), lambda i: (0, 0, 0)),
        ],
        out_specs=[
            pl.BlockSpec((R, C), lambda i: (i, 0)),
            pl.BlockSpec((R, 8), lambda i: (i, 0)),
            pl.BlockSpec((8, 128), lambda i: (0, 0)),
            pl.BlockSpec((8, 128), lambda i: (0, 0)),
        ],
        out_shape=[
            jax.ShapeDtypeStruct((NROWS, C), jnp.float32),
            jax.ShapeDtypeStruct((NROWS, 8), jnp.int32),
            jax.ShapeDtypeStruct((8, 128), jnp.float32),
            jax.ShapeDtypeStruct((8, 128), jnp.float32),
        ],
        scratch_shapes=[pltpu.VMEM((8, K), jnp.float32)],
    )(xf, codebooks, cbsq, cb3)
    qo = qo_flat.reshape(BB, TT, C).transpose(0, 2, 1)
    indices = idx8[:, :NQ].reshape(BB, TT, NQ)
    losses = loss8[:NQ, 0] / (NROWS * C)
    perp = perp8[:NQ, 0]
    return qo, indices, losses, perp


# speedup vs baseline: 1.8795x; 1.2060x over previous
import jax
import jax.numpy as jnp
from jax import lax
from jax.experimental import pallas as pl
from jax.experimental.pallas import tpu as pltpu

NQ = 6
K = 1024
C = 256
BB = 64
TT = 576
NROWS = BB * TT
R = 256
NH = 2
RH = R // NH
NBLK = NROWS // R


def _vq_kernel(xf_ref, cb_ref, cbsq_ref, cb3_ref,
               qo_ref, idx_ref, loss_ref, perp_ref, counts):
    i = pl.program_id(0)
    ii = lax.broadcasted_iota(jnp.int32, (RH, K), 1)
    dn = (((1,), (0,)), ((), ()))
    res = [xf_ref[h * RH:(h + 1) * RH, :] for h in range(NH)]
    qac = [jnp.zeros((RH, C), jnp.float32) for _ in range(NH)]
    idx_cols = [[] for _ in range(NH)]
    loss_rows = []
    count_rows = []
    for q in range(NQ):
        cb = cb_ref[q]
        cbsq = cbsq_ref[q]
        cb3 = cb3_ref[q]
        counts_h = []
        sq_h = []
        for h in range(NH):
            r_ = res[h]
            rsq = jnp.sum(r_ * r_, axis=1, keepdims=True)
            cross = lax.dot_general(r_, cb, (((1,), (1,)), ((), ())),
                                    preferred_element_type=jnp.float32)
            d = rsq - 2.0 * cross + cbsq
            idxc = jnp.argmin(d, axis=1, keepdims=True).astype(jnp.int32)
            ohf = (ii == idxc).astype(jnp.float32)
            oh = ohf.astype(jnp.bfloat16)
            oh3 = jnp.concatenate([oh, oh, oh], axis=1)
            x_d = lax.dot_general(oh3, cb3, dn,
                                  preferred_element_type=jnp.float32)
            r_ = r_ - x_d
            res[h] = r_
            qac[h] = qac[h] + x_d
            counts_h.append(jnp.sum(ohf, axis=0, keepdims=True))
            sq_h.append(jnp.sum(r_ * r_))
            idx_cols[h].append(idxc)
        count_rows.append(counts_h[0] + counts_h[1])
        loss_rows.append(jnp.full((1, 128), sq_h[0] + sq_h[1], jnp.float32))

    qo_ref[...] = jnp.concatenate(qac, axis=0)
    idx_ref[...] = jnp.concatenate(
        [jnp.concatenate(cols + [jnp.zeros((RH, 8 - NQ), jnp.int32)], axis=1)
         for cols in idx_cols], axis=0)
    loss_blk = jnp.concatenate(
        loss_rows + [jnp.zeros((8 - NQ, 128), jnp.float32)], axis=0)
    counts_blk = jnp.concatenate(
        count_rows + [jnp.zeros((8 - NQ, K), jnp.float32)], axis=0)

    @pl.when(i == 0)
    def _():
        counts[...] = counts_blk
        loss_ref[...] = loss_blk

    @pl.when(i > 0)
    def _():
        counts[...] += counts_blk
        loss_ref[...] += loss_blk

    @pl.when(i == NBLK - 1)
    def _():
        prob = counts[...] * (1.0 / NROWS)
        plog = prob * jnp.log(prob + 1e-7)
        s = jnp.sum(plog, axis=1, keepdims=True)
        perp_ref[...] = jnp.broadcast_to(jnp.exp(-s), (8, 128))


def kernel(x, codebooks):
    xf = x.transpose(0, 2, 1).reshape(NROWS, C)
    cbsq = jnp.sum(codebooks ** 2, axis=-1).reshape(NQ, 1, K)
    bits = lax.bitcast_convert_type(codebooks, jnp.int32)
    hi_f = lax.bitcast_convert_type(bits & jnp.int32(-65536), jnp.float32)
    rem = codebooks - hi_f
    rbits = lax.bitcast_convert_type(rem, jnp.int32)
    mid_f = lax.bitcast_convert_type(rbits & jnp.int32(-65536), jnp.float32)
    lo_f = rem - mid_f
    cb_hi = hi_f.astype(jnp.bfloat16)
    cb_mid = mid_f.astype(jnp.bfloat16)
    cb_lo = lo_f.astype(jnp.bfloat16)
    cb3 = jnp.concatenate([cb_lo, cb_mid, cb_hi], axis=1)
    qo_flat, idx8, loss8, perp8 = pl.pallas_call(
        _vq_kernel,
        grid=(NBLK,),
        in_specs=[
            pl.BlockSpec((R, C), lambda i: (i, 0)),
            pl.BlockSpec((NQ, K, C), lambda i: (0, 0, 0)),
            pl.BlockSpec((NQ, 1, K), lambda i: (0, 0, 0)),
            pl.BlockSpec((NQ, 3 * K, C), lambda i: (0, 0, 0)),
        ],
        out_specs=[
            pl.BlockSpec((R, C), lambda i: (i, 0)),
            pl.BlockSpec((R, 8), lambda i: (i, 0)),
            pl.BlockSpec((8, 128), lambda i: (0, 0)),
            pl.BlockSpec((8, 128), lambda i: (0, 0)),
        ],
        out_shape=[
            jax.ShapeDtypeStruct((NROWS, C), jnp.float32),
            jax.ShapeDtypeStruct((NROWS, 8), jnp.int32),
            jax.ShapeDtypeStruct((8, 128), jnp.float32),
            jax.ShapeDtypeStruct((8, 128), jnp.float32),
        ],
        scratch_shapes=[pltpu.VMEM((8, K), jnp.float32)],
    )(xf, codebooks, cbsq, cb3)
    qo = qo_flat.reshape(BB, TT, C).transpose(0, 2, 1)
    indices = idx8[:, :NQ].reshape(BB, TT, NQ)
    losses = loss8[:NQ, 0] / (NROWS * C)
    perp = perp8[:NQ, 0]
    return qo, indices, losses, perp
